# baseline (speedup 1.0000x reference)
import functools

import jax
import jax.numpy as jnp
from jax import lax
from jax.experimental import pallas as pl
from jax.experimental.pallas import tpu as pltpu
from jax.experimental.pallas import tpu_sc as plsc

VOCAB = 100000
H = 768
B = 1024
S = 50
SA = 48
SB = 2
SP = 56
NW = 32
NSPLIT = 4
BQ = B // NSPLIT
BPW = BQ // NW
DEPTH = 2
EPS = 1e-5
TCB = 8


def _sc_gather(table_hbm, idx_hbm, out_hbm, idx_v, bufs_a, bufs_b, sems):
    wid = lax.axis_index("s") * 2 + lax.axis_index("c")
    base = wid * BPW
    pltpu.sync_copy(idx_hbm.at[pl.ds(base * SP, BPW * SP)], idx_v)

    def g_start(t, b):
        pltpu.async_copy(table_hbm.at[idx_v.at[pl.ds(t * SP, SA)]],
                         bufs_a[b], sems[b])
        pltpu.async_copy(table_hbm.at[idx_v.at[pl.ds(t * SP + SA, SB)]],
                         bufs_b[b], sems[b])

    def g_wait(t, b):
        pltpu.make_async_copy(table_hbm.at[idx_v.at[pl.ds(t * SP, SA)]],
                              bufs_a[b], sems[b]).wait()
        pltpu.make_async_copy(table_hbm.at[idx_v.at[pl.ds(t * SP + SA, SB)]],
                              bufs_b[b], sems[b]).wait()

    for b in range(DEPTH):
        g_start(b, b)

    def outer(g, _):
        for b in range(DEPTH):
            t = g * DEPTH + b
            g_wait(t, b)
            pltpu.sync_copy(bufs_a[b], out_hbm.at[base + t, pl.ds(0, SA)])
            pltpu.sync_copy(bufs_b[b], out_hbm.at[base + t, pl.ds(SA, SB)])
            nxt = t + DEPTH

            @pl.when(nxt < BPW)
            def _():
                g_start(nxt, b)
        return _
    lax.fori_loop(0, BPW // DEPTH, outer, None)


def _gather_rows(fixed_ans_emb, idx_arranged):
    mesh = plsc.VectorSubcoreMesh(core_axis_name="c", subcore_axis_name="s")
    fn = pl.kernel(
        _sc_gather,
        mesh=mesh,
        out_type=jax.ShapeDtypeStruct((BQ, S, H), jnp.float32),
        scratch_types=[
            pltpu.VMEM((BPW * SP,), jnp.int32),
            [pltpu.VMEM((SA, H), jnp.float32) for _ in range(DEPTH)],
            [pltpu.VMEM((SB, H), jnp.float32) for _ in range(DEPTH)],
            [pltpu.SemaphoreType.DMA for _ in range(DEPTH)],
        ],
    )
    return fn(fixed_ans_emb, idx_arranged)


def _tc_ln(gath_ref, pos_ref, gpos_ref, bpos_ref, gans_ref, bans_ref,
           out_ref, padd_ref):
    @pl.when(pl.program_id(0) == 0)
    def _():
        p = pos_ref[...]
        m = jnp.mean(p, axis=1, keepdims=True)
        d = p - m
        v = jnp.mean(d * d, axis=1, keepdims=True)
        padd_ref[...] = (d * lax.rsqrt(v + EPS) * gpos_ref[...]
                         + bpos_ref[...] + bans_ref[...])

    x = gath_ref[...]
    m = jnp.mean(x, axis=2, keepdims=True)
    v = jnp.mean(x * x, axis=2, keepdims=True) - m * m
    out_ref[...] = ((x - m) * lax.rsqrt(v + EPS) * gans_ref[...][None]
                    + padd_ref[...][None])


def _ln_posadd(gathered, pos_table, gpos2, bpos2, gans2, bans2):
    return pl.pallas_call(
        _tc_ln,
        grid=(BQ // TCB,),
        in_specs=[
            pl.BlockSpec((TCB, S, H), lambda i: (i, 0, 0)),
            pl.BlockSpec((S, H), lambda i: (0, 0)),
            pl.BlockSpec((1, H), lambda i: (0, 0)),
            pl.BlockSpec((1, H), lambda i: (0, 0)),
            pl.BlockSpec((1, H), lambda i: (0, 0)),
            pl.BlockSpec((1, H), lambda i: (0, 0)),
        ],
        out_specs=pl.BlockSpec((TCB, S, H), lambda i: (i, 0, 0)),
        out_shape=jax.ShapeDtypeStruct((BQ, S, H), jnp.float32),
        scratch_shapes=[pltpu.VMEM((S, H), jnp.float32)],
    )(gathered, pos_table, gpos2, bpos2, gans2, bans2)


@jax.jit
def _prev_embedding(fixed_ans_emb, idx_padded, pos_table, ln_pos_g, ln_pos_b,
                    ln_ans_g, ln_ans_b):
    gpos2 = ln_pos_g.reshape(1, H)
    bpos2 = ln_pos_b.reshape(1, H)
    gans2 = ln_ans_g.reshape(1, H)
    bans2 = ln_ans_b.reshape(1, H)
    out = jnp.zeros((B, S, H), jnp.float32)
    for q in range(NSPLIT):
        idx_q = idx_padded[q * BQ:(q + 1) * BQ].reshape(-1)
        g_q = _gather_rows(fixed_ans_emb, idx_q)
        y_q = _ln_posadd(g_q, pos_table, gpos2, bpos2, gans2, bans2)
        out = lax.dynamic_update_slice(out, y_q, (q * BQ, 0, 0))
    return out


def kernel(fixed_ans_emb, prev_inds, pos_table, ln_pos_g, ln_pos_b,
           ln_ans_g, ln_ans_b):
    idx = prev_inds.astype(jnp.int32)
    idx_padded = jnp.pad(idx, ((0, 0), (0, SP - S)))
    return _prev_embedding(fixed_ans_emb, idx_padded, pos_table, ln_pos_g,
                           ln_pos_b, ln_ans_g, ln_ans_b)


# speedup vs baseline: 1.1205x; 1.1205x over previous
import functools

import jax
import jax.numpy as jnp
from jax import lax
from jax.experimental import pallas as pl
from jax.experimental.pallas import tpu as pltpu
from jax.experimental.pallas import tpu_sc as plsc

VOCAB = 100000
H = 768
B = 1024
S = 50
SA = 48
SB = 2
SP = 56
NW = 32
NSPLIT = 4
BQ = B // NSPLIT
BPW = BQ // NW
DEPTH = 2
EPS = 1e-5
TCB = 8


def _sc_gather(table_hbm, idx_hbm, out_hbm, idx_v, bufs_a, bufs_b, sems):
    wid = lax.axis_index("s") * 2 + lax.axis_index("c")
    base = wid * BPW
    pltpu.sync_copy(idx_hbm.at[pl.ds(base * SP, BPW * SP)], idx_v)

    def g_start(t, b):
        pltpu.async_copy(table_hbm.at[idx_v.at[pl.ds(t * SP, SA)]],
                         bufs_a[b], sems[b])
        pltpu.async_copy(table_hbm.at[idx_v.at[pl.ds(t * SP + SA, SB)]],
                         bufs_b[b], sems[b])

    def g_wait(t, b):
        pltpu.make_async_copy(table_hbm.at[idx_v.at[pl.ds(t * SP, SA)]],
                              bufs_a[b], sems[b]).wait()
        pltpu.make_async_copy(table_hbm.at[idx_v.at[pl.ds(t * SP + SA, SB)]],
                              bufs_b[b], sems[b]).wait()

    for b in range(DEPTH):
        g_start(b, b)

    def outer(g, _):
        for b in range(DEPTH):
            t = g * DEPTH + b
            g_wait(t, b)
            pltpu.sync_copy(bufs_a[b], out_hbm.at[base + t, pl.ds(0, SA)])
            pltpu.sync_copy(bufs_b[b], out_hbm.at[base + t, pl.ds(SA, SB)])
            nxt = t + DEPTH

            @pl.when(nxt < BPW)
            def _():
                g_start(nxt, b)
        return _
    lax.fori_loop(0, BPW // DEPTH, outer, None)


def _gather_rows(fixed_ans_emb, idx_arranged):
    mesh = plsc.VectorSubcoreMesh(core_axis_name="c", subcore_axis_name="s")
    fn = pl.kernel(
        _sc_gather,
        mesh=mesh,
        out_type=jax.ShapeDtypeStruct((BQ, S, H), jnp.float32),
        scratch_types=[
            pltpu.VMEM((BPW * SP,), jnp.int32),
            [pltpu.VMEM((SA, H), jnp.float32) for _ in range(DEPTH)],
            [pltpu.VMEM((SB, H), jnp.float32) for _ in range(DEPTH)],
            [pltpu.SemaphoreType.DMA for _ in range(DEPTH)],
        ],
    )
    return fn(fixed_ans_emb, idx_arranged)


def _tc_ln(gath_ref, pos_ref, gpos_ref, bpos_ref, gans_ref, bans_ref,
           out_ref, padd_ref):
    @pl.when(pl.program_id(0) == 0)
    def _():
        p = pos_ref[...]
        m = jnp.mean(p, axis=1, keepdims=True)
        d = p - m
        v = jnp.mean(d * d, axis=1, keepdims=True)
        padd_ref[...] = (d * lax.rsqrt(v + EPS) * gpos_ref[...]
                         + bpos_ref[...] + bans_ref[...])

    x = gath_ref[...]
    m = jnp.mean(x, axis=2, keepdims=True)
    v = jnp.mean(x * x, axis=2, keepdims=True) - m * m
    out_ref[...] = ((x - m) * lax.rsqrt(v + EPS) * gans_ref[...][None]
                    + padd_ref[...][None])


def _ln_posadd(gathered, pos_table, gpos2, bpos2, gans2, bans2):
    return pl.pallas_call(
        _tc_ln,
        grid=(BQ // TCB,),
        in_specs=[
            pl.BlockSpec((TCB, S, H), lambda i: (i, 0, 0)),
            pl.BlockSpec((S, H), lambda i: (0, 0)),
            pl.BlockSpec((1, H), lambda i: (0, 0)),
            pl.BlockSpec((1, H), lambda i: (0, 0)),
            pl.BlockSpec((1, H), lambda i: (0, 0)),
            pl.BlockSpec((1, H), lambda i: (0, 0)),
        ],
        out_specs=pl.BlockSpec((TCB, S, H), lambda i: (i, 0, 0)),
        out_shape=jax.ShapeDtypeStruct((BQ, S, H), jnp.float32),
        scratch_shapes=[pltpu.VMEM((S, H), jnp.float32)],
    )(gathered, pos_table, gpos2, bpos2, gans2, bans2)


@jax.jit
def _prev_embedding(fixed_ans_emb, idx_padded, pos_table, ln_pos_g, ln_pos_b,
                    ln_ans_g, ln_ans_b):
    gpos2 = ln_pos_g.reshape(1, H)
    bpos2 = ln_pos_b.reshape(1, H)
    gans2 = ln_ans_g.reshape(1, H)
    bans2 = ln_ans_b.reshape(1, H)
    ys = []
    for q in range(NSPLIT):
        idx_q = idx_padded[q * BQ:(q + 1) * BQ].reshape(-1)
        g_q = _gather_rows(fixed_ans_emb, idx_q)
        ys.append(_ln_posadd(g_q, pos_table, gpos2, bpos2, gans2, bans2))
    return jnp.concatenate(ys, axis=0)


def kernel(fixed_ans_emb, prev_inds, pos_table, ln_pos_g, ln_pos_b,
           ln_ans_g, ln_ans_b):
    idx = prev_inds.astype(jnp.int32)
    idx_padded = jnp.pad(idx, ((0, 0), (0, SP - S)))
    return _prev_embedding(fixed_ans_emb, idx_padded, pos_table, ln_pos_g,
                           ln_pos_b, ln_ans_g, ln_ans_b)


# speedup vs baseline: 1.7565x; 1.5677x over previous
import functools

import jax
import jax.numpy as jnp
from jax import lax
from jax.experimental import pallas as pl
from jax.experimental.pallas import tpu as pltpu
from jax.experimental.pallas import tpu_sc as plsc

VOCAB = 100000
H = 768
B = 1024
S = 50
N = B * S
NW = 32
PER_W = N // NW
CH = 32
BCH = B // CH
NCH = PER_W // CH
DEPTH = 2
EPS = 1e-5
TCN = 256


def _sc_gather(table_hbm, idx_hbm, out_hbm, idx_v, bufs, sems):
    wid = lax.axis_index("s") * 2 + lax.axis_index("c")
    base = wid * NCH
    pltpu.sync_copy(idx_hbm.at[pl.ds(base * CH, NCH * CH)], idx_v)

    def g_start(t, b):
        pltpu.async_copy(table_hbm.at[idx_v.at[pl.ds(t * CH, CH)]],
                         bufs[b], sems[b])

    def g_wait(t, b):
        pltpu.make_async_copy(table_hbm.at[idx_v.at[pl.ds(t * CH, CH)]],
                              bufs[b], sems[b]).wait()

    def g_out(t, b):
        gt = base + t
        s = gt // BCH
        b0 = (gt % BCH) * CH
        pltpu.sync_copy(bufs[b], out_hbm.at[s, pl.ds(b0, CH)])

    for b in range(DEPTH):
        g_start(b, b)

    def outer(g, _):
        for b in range(DEPTH):
            t = g * DEPTH + b
            g_wait(t, b)
            g_out(t, b)
            nxt = t + DEPTH

            @pl.when(nxt < NCH)
            def _():
                g_start(nxt, b)
        return _
    lax.fori_loop(0, NCH // DEPTH, outer, None)


def _gather_rows(fixed_ans_emb, idx_pm):
    mesh = plsc.VectorSubcoreMesh(core_axis_name="c", subcore_axis_name="s")
    fn = pl.kernel(
        _sc_gather,
        mesh=mesh,
        out_type=jax.ShapeDtypeStruct((S, B, H), jnp.float32),
        scratch_types=[
            pltpu.VMEM((PER_W,), jnp.int32),
            [pltpu.VMEM((CH, H), jnp.float32) for _ in range(DEPTH)],
            [pltpu.SemaphoreType.DMA for _ in range(DEPTH)],
        ],
    )
    return fn(fixed_ans_emb, idx_pm)


def _tc_ln(gath_ref, pos_ref, gpos_ref, bpos_ref, gans_ref, bans_ref,
           out_ref, padd_ref):
    si = pl.program_id(0)

    @pl.when((si == 0) & (pl.program_id(1) == 0))
    def _():
        p = pos_ref[...]
        m = jnp.mean(p, axis=1, keepdims=True)
        d = p - m
        v = jnp.mean(d * d, axis=1, keepdims=True)
        padd_ref[...] = (d * lax.rsqrt(v + EPS) * gpos_ref[...]
                         + bpos_ref[...] + bans_ref[...])

    x = gath_ref[...]
    m = jnp.mean(x, axis=2, keepdims=True)
    v = jnp.mean(x * x, axis=2, keepdims=True) - m * m
    out_ref[...] = ((x - m) * lax.rsqrt(v + EPS) * gans_ref[...][None]
                    + padd_ref[pl.ds(si, 1)][None])


def _ln_posadd(gathered, pos_table, gpos2, bpos2, gans2, bans2):
    return pl.pallas_call(
        _tc_ln,
        grid=(S, B // TCN),
        in_specs=[
            pl.BlockSpec((1, TCN, H), lambda i, j: (i, j, 0)),
            pl.BlockSpec((S, H), lambda i, j: (0, 0)),
            pl.BlockSpec((1, H), lambda i, j: (0, 0)),
            pl.BlockSpec((1, H), lambda i, j: (0, 0)),
            pl.BlockSpec((1, H), lambda i, j: (0, 0)),
            pl.BlockSpec((1, H), lambda i, j: (0, 0)),
        ],
        out_specs=pl.BlockSpec((1, TCN, H), lambda i, j: (i, j, 0)),
        out_shape=jax.ShapeDtypeStruct((S, B, H), jnp.float32),
        scratch_shapes=[pltpu.VMEM((S, H), jnp.float32)],
    )(gathered, pos_table, gpos2, bpos2, gans2, bans2)


@jax.jit
def _prev_embedding(fixed_ans_emb, idx_pm, pos_table, ln_pos_g, ln_pos_b,
                    ln_ans_g, ln_ans_b):
    gathered = _gather_rows(fixed_ans_emb, idx_pm)
    z = _ln_posadd(gathered, pos_table, ln_pos_g.reshape(1, H),
                   ln_pos_b.reshape(1, H), ln_ans_g.reshape(1, H),
                   ln_ans_b.reshape(1, H))
    return jnp.transpose(z, (1, 0, 2))


def kernel(fixed_ans_emb, prev_inds, pos_table, ln_pos_g, ln_pos_b,
           ln_ans_g, ln_ans_b):
    idx_pm = prev_inds.astype(jnp.int32).T.reshape(-1)
    return _prev_embedding(fixed_ans_emb, idx_pm, pos_table, ln_pos_g,
                           ln_pos_b, ln_ans_g, ln_ans_b)


# speedup vs baseline: 2.1155x; 1.2044x over previous
import functools

import jax
import jax.numpy as jnp
from jax import lax
from jax.experimental import pallas as pl
from jax.experimental.pallas import tpu as pltpu
from jax.experimental.pallas import tpu_sc as plsc

VOCAB = 100000
H = 768
B = 1024
S = 50
N = B * S
NW = 32
PER_W = N // NW
CH = 32
BCH = B // CH
NCH = PER_W // CH
DEPTH = 2
EPS = 1e-5
TCN = 512


def _sc_gather(table_hbm, idx_hbm, out_hbm, idx_v, bufs, sems):
    wid = lax.axis_index("s") * 2 + lax.axis_index("c")
    base = wid * NCH
    pltpu.sync_copy(idx_hbm.at[pl.ds(base * CH, NCH * CH)], idx_v)

    def g_start(t, b):
        pltpu.async_copy(table_hbm.at[idx_v.at[pl.ds(t * CH, CH)]],
                         bufs[b], sems[b])

    def g_wait(t, b):
        pltpu.make_async_copy(table_hbm.at[idx_v.at[pl.ds(t * CH, CH)]],
                              bufs[b], sems[b]).wait()

    def g_out(t, b):
        gt = base + t
        s = gt // BCH
        b0 = (gt % BCH) * CH
        pltpu.sync_copy(bufs[b], out_hbm.at[s, pl.ds(b0, CH)])

    for b in range(DEPTH):
        g_start(b, b)

    def outer(g, _):
        for b in range(DEPTH):
            t = g * DEPTH + b
            g_wait(t, b)
            g_out(t, b)
            nxt = t + DEPTH

            @pl.when(nxt < NCH)
            def _():
                g_start(nxt, b)
        return _
    lax.fori_loop(0, NCH // DEPTH, outer, None)


def _gather_rows(fixed_ans_emb, idx_pm):
    mesh = plsc.VectorSubcoreMesh(core_axis_name="c", subcore_axis_name="s")
    fn = pl.kernel(
        _sc_gather,
        mesh=mesh,
        out_type=jax.ShapeDtypeStruct((S, B, H), jnp.float32),
        scratch_types=[
            pltpu.VMEM((PER_W,), jnp.int32),
            [pltpu.VMEM((CH, H), jnp.float32) for _ in range(DEPTH)],
            [pltpu.SemaphoreType.DMA for _ in range(DEPTH)],
        ],
    )
    return fn(fixed_ans_emb, idx_pm)


def _tc_ln(gath_ref, pos_ref, gpos_ref, bpos_ref, gans_ref, bans_ref,
           out_ref, padd_ref):
    si = pl.program_id(0)

    @pl.when((si == 0) & (pl.program_id(1) == 0))
    def _():
        p = pos_ref[...]
        m = jnp.mean(p, axis=1, keepdims=True)
        d = p - m
        v = jnp.mean(d * d, axis=1, keepdims=True)
        padd_ref[...] = (d * lax.rsqrt(v + EPS) * gpos_ref[...]
                         + bpos_ref[...] + bans_ref[...])

    x = gath_ref[...]
    m = jnp.mean(x, axis=2, keepdims=True)
    v = jnp.mean(x * x, axis=2, keepdims=True) - m * m
    out_ref[...] = ((x - m) * lax.rsqrt(v + EPS) * gans_ref[...][None]
                    + padd_ref[pl.ds(si, 1)][None])


def _ln_posadd(gathered, pos_table, gpos2, bpos2, gans2, bans2):
    return pl.pallas_call(
        _tc_ln,
        grid=(S, B // TCN),
        in_specs=[
            pl.BlockSpec((1, TCN, H), lambda i, j: (i, j, 0)),
            pl.BlockSpec((S, H), lambda i, j: (0, 0)),
            pl.BlockSpec((1, H), lambda i, j: (0, 0)),
            pl.BlockSpec((1, H), lambda i, j: (0, 0)),
            pl.BlockSpec((1, H), lambda i, j: (0, 0)),
            pl.BlockSpec((1, H), lambda i, j: (0, 0)),
        ],
        out_specs=pl.BlockSpec((1, TCN, H), lambda i, j: (i, j, 0)),
        out_shape=jax.ShapeDtypeStruct((S, B, H), jnp.float32),
        scratch_shapes=[pltpu.VMEM((S, H), jnp.float32)],
    )(gathered, pos_table, gpos2, bpos2, gans2, bans2)


@jax.jit
def _prev_embedding(fixed_ans_emb, idx_pm, pos_table, ln_pos_g, ln_pos_b,
                    ln_ans_g, ln_ans_b):
    gathered = _gather_rows(fixed_ans_emb, idx_pm)
    z = _ln_posadd(gathered, pos_table, ln_pos_g.reshape(1, H),
                   ln_pos_b.reshape(1, H), ln_ans_g.reshape(1, H),
                   ln_ans_b.reshape(1, H))
    return jnp.transpose(z, (1, 0, 2))


def kernel(fixed_ans_emb, prev_inds, pos_table, ln_pos_g, ln_pos_b,
           ln_ans_g, ln_ans_b):
    idx_pm = prev_inds.astype(jnp.int32).T.reshape(-1)
    return _prev_embedding(fixed_ans_emb, idx_pm, pos_table, ln_pos_g,
                           ln_pos_b, ln_ans_g, ln_ans_b)


# speedup vs baseline: 2.3481x; 1.1099x over previous
import functools

import jax
import jax.numpy as jnp
from jax import lax
from jax.experimental import pallas as pl
from jax.experimental.pallas import tpu as pltpu
from jax.experimental.pallas import tpu_sc as plsc

VOCAB = 100000
H = 768
B = 1024
S = 50
N = B * S
NW = 32
PER_W = N // NW
CH = 32
BCH = B // CH
NCH = PER_W // CH
DEPTH = 2
EPS = 1e-5
TCN = 1024


def _sc_gather(table_hbm, idx_hbm, out_hbm, idx_v, bufs, sems):
    wid = lax.axis_index("s") * 2 + lax.axis_index("c")
    base = wid * NCH
    pltpu.sync_copy(idx_hbm.at[pl.ds(base * CH, NCH * CH)], idx_v)

    def g_start(t, b):
        pltpu.async_copy(table_hbm.at[idx_v.at[pl.ds(t * CH, CH)]],
                         bufs[b], sems[b])

    def g_wait(t, b):
        pltpu.make_async_copy(table_hbm.at[idx_v.at[pl.ds(t * CH, CH)]],
                              bufs[b], sems[b]).wait()

    def g_out(t, b):
        gt = base + t
        s = gt // BCH
        b0 = (gt % BCH) * CH
        pltpu.sync_copy(bufs[b], out_hbm.at[s, pl.ds(b0, CH)])

    for b in range(DEPTH):
        g_start(b, b)

    def outer(g, _):
        for b in range(DEPTH):
            t = g * DEPTH + b
            g_wait(t, b)
            g_out(t, b)
            nxt = t + DEPTH

            @pl.when(nxt < NCH)
            def _():
                g_start(nxt, b)
        return _
    lax.fori_loop(0, NCH // DEPTH, outer, None)


def _gather_rows(fixed_ans_emb, idx_pm):
    mesh = plsc.VectorSubcoreMesh(core_axis_name="c", subcore_axis_name="s")
    fn = pl.kernel(
        _sc_gather,
        mesh=mesh,
        out_type=jax.ShapeDtypeStruct((S, B, H), jnp.float32),
        scratch_types=[
            pltpu.VMEM((PER_W,), jnp.int32),
            [pltpu.VMEM((CH, H), jnp.float32) for _ in range(DEPTH)],
            [pltpu.SemaphoreType.DMA for _ in range(DEPTH)],
        ],
    )
    return fn(fixed_ans_emb, idx_pm)


def _tc_ln(gath_ref, pos_ref, gpos_ref, bpos_ref, gans_ref, bans_ref,
           out_ref, padd_ref):
    si = pl.program_id(0)

    @pl.when((si == 0) & (pl.program_id(1) == 0))
    def _():
        p = pos_ref[...]
        m = jnp.mean(p, axis=1, keepdims=True)
        d = p - m
        v = jnp.mean(d * d, axis=1, keepdims=True)
        padd_ref[...] = (d * lax.rsqrt(v + EPS) * gpos_ref[...]
                         + bpos_ref[...] + bans_ref[...])

    x = gath_ref[...]
    m = jnp.mean(x, axis=2, keepdims=True)
    v = jnp.mean(x * x, axis=2, keepdims=True) - m * m
    out_ref[...] = ((x - m) * lax.rsqrt(v + EPS) * gans_ref[...][None]
                    + padd_ref[pl.ds(si, 1)][None])


def _ln_posadd(gathered, pos_table, gpos2, bpos2, gans2, bans2):
    return pl.pallas_call(
        _tc_ln,
        grid=(S, B // TCN),
        in_specs=[
            pl.BlockSpec((1, TCN, H), lambda i, j: (i, j, 0)),
            pl.BlockSpec((S, H), lambda i, j: (0, 0)),
            pl.BlockSpec((1, H), lambda i, j: (0, 0)),
            pl.BlockSpec((1, H), lambda i, j: (0, 0)),
            pl.BlockSpec((1, H), lambda i, j: (0, 0)),
            pl.BlockSpec((1, H), lambda i, j: (0, 0)),
        ],
        out_specs=pl.BlockSpec((1, TCN, H), lambda i, j: (i, j, 0)),
        out_shape=jax.ShapeDtypeStruct((S, B, H), jnp.float32),
        scratch_shapes=[pltpu.VMEM((S, H), jnp.float32)],
    )(gathered, pos_table, gpos2, bpos2, gans2, bans2)


@jax.jit
def _prev_embedding(fixed_ans_emb, idx_pm, pos_table, ln_pos_g, ln_pos_b,
                    ln_ans_g, ln_ans_b):
    gathered = _gather_rows(fixed_ans_emb, idx_pm)
    z = _ln_posadd(gathered, pos_table, ln_pos_g.reshape(1, H),
                   ln_pos_b.reshape(1, H), ln_ans_g.reshape(1, H),
                   ln_ans_b.reshape(1, H))
    return jnp.transpose(z, (1, 0, 2))


def kernel(fixed_ans_emb, prev_inds, pos_table, ln_pos_g, ln_pos_b,
           ln_ans_g, ln_ans_b):
    idx_pm = prev_inds.astype(jnp.int32).T.reshape(-1)
    return _prev_embedding(fixed_ans_emb, idx_pm, pos_table, ln_pos_g,
                           ln_pos_b, ln_ans_g, ln_ans_b)
